# baseline (speedup 1.0000x reference)
import jax
import jax.numpy as jnp
from jax import lax
from jax.experimental import pallas as pl
from jax.experimental.pallas import tpu as pltpu
from jax.experimental.pallas import tpu_sc as plsc

_NUM_CORES = 2
_NUM_SUBCORES = 16
_CHUNK = 32
_NBUF = 5


def kernel(x, embed_weight):
    b, s = x.shape
    n = b * s
    v, d = embed_weight.shape
    nw = _NUM_CORES * _NUM_SUBCORES
    bw = n // nw
    nchunk = bw // _CHUNK

    mesh = plsc.VectorSubcoreMesh(core_axis_name="c", subcore_axis_name="s")

    @pl.kernel(
        out_type=jax.ShapeDtypeStruct((b, s, d), embed_weight.dtype),
        mesh=mesh,
        scratch_types=[
            pltpu.VMEM((bw,), jnp.int32),
        ] + [pltpu.VMEM((_CHUNK, d), jnp.float32) for _ in range(_NBUF)] + [
            pltpu.SemaphoreType.DMA,
            pltpu.SemaphoreType.DMA,
        ],
    )
    def k(table_hbm, i_hbm, o_hbm, idx_v, *rest):
        bufs = rest[:_NBUF]
        gsem, wsem = rest[_NBUF:]
        wid = lax.axis_index("s") * _NUM_CORES + lax.axis_index("c")
        base = wid * bw
        bi = base // s
        col0 = base % s
        pltpu.sync_copy(i_hbm.at[bi, pl.ds(col0, bw)], idx_v)

        def start_gather(c):
            return pltpu.async_copy(
                table_hbm.at[idx_v.at[pl.ds(c * _CHUNK, _CHUNK)]],
                bufs[c % _NBUF], gsem)

        gathers = [None] * nchunk
        writes = [None] * nchunk
        for c in range(_NBUF - 1):
            gathers[c] = start_gather(c)
        for c in range(nchunk):
            gathers[c].wait()
            writes[c] = pltpu.async_copy(
                bufs[c % _NBUF],
                o_hbm.at[bi, pl.ds(col0 + c * _CHUNK, _CHUNK)], wsem)
            if c + _NBUF - 1 < nchunk:
                if c >= 1:
                    writes[c - 1].wait()
                gathers[c + _NBUF - 1] = start_gather(c + _NBUF - 1)
        for c in range(max(0, nchunk - _NBUF), nchunk):
            writes[c].wait()

    return k(embed_weight, x.astype(jnp.int32))
